# Initial kernel scaffold; baseline (speedup 1.0000x reference)
import jax
import jax.numpy as jnp
from jax.experimental import pallas as pl


def kernel(title_ids, token_ids, title_table, token_table):
    raise NotImplementedError("write your pallas kernel here")


# speedup vs baseline: 8.4240x; 8.4240x over previous
import functools

import jax
import jax.numpy as jnp
from jax import lax
from jax.experimental import pallas as pl
from jax.experimental.pallas import tpu as pltpu
from jax.experimental.pallas import tpu_sc as plsc

D = 64
SEQ = 20
B = 16384
NC = 2
NS = 16
NW = NC * NS
ROWS_PER_W = B // NW
CHUNK = 32
NCHUNK = ROWS_PER_W // CHUNK
TOK_PER_CHUNK = CHUNK * SEQ
IDX_PER_STREAM = 128
NSTREAM = TOK_PER_CHUNK // IDX_PER_STREAM


def _lanes():
    return lax.broadcasted_iota(jnp.int32, (16,), 0)


_mesh = plsc.VectorSubcoreMesh(core_axis_name="c", subcore_axis_name="s")


@functools.partial(
    pl.kernel,
    mesh=_mesh,
    compiler_params=pltpu.CompilerParams(needs_layout_passes=False,
                                         use_tc_tiling_on_sc=False),
    out_type=jax.ShapeDtypeStruct((B, 2 * D), jnp.float32),
    scratch_types=[
        pltpu.VMEM((TOK_PER_CHUNK,), jnp.int32),
        pltpu.VMEM((CHUNK,), jnp.int32),
        pltpu.VMEM((TOK_PER_CHUNK, D), jnp.float32),
        pltpu.VMEM((CHUNK, D), jnp.float32),
        pltpu.VMEM((CHUNK, 2 * D), jnp.float32),
        pltpu.VMEM((1, D), jnp.float32),
        pltpu.SemaphoreType.DMA,
        pltpu.SemaphoreType.DMA,
    ],
)
def _movie_sc_kernel(title_ids, tok_flat, title_table, token_table, out,
                     tok_v, tid_v, rows_v, trows_v, outb_v,
                     row0_v, sem_tok, sem_title):
    wid = lax.axis_index("s") * NC + lax.axis_index("c")
    base0 = wid * ROWS_PER_W

    pltpu.sync_copy(token_table.at[pl.ds(0, 1), :], row0_v)

    def chunk_body(ci, carry):
        base = base0 + ci * CHUNK
        pltpu.sync_copy(tok_flat.at[pl.ds(base * SEQ, TOK_PER_CHUNK)], tok_v)
        pltpu.sync_copy(title_ids.at[pl.ds(base, CHUNK)], tid_v)

        tok_copies = [
            pltpu.async_copy(
                token_table.at[tok_v.at[pl.ds(j * IDX_PER_STREAM,
                                              IDX_PER_STREAM)]],
                rows_v.at[pl.ds(j * IDX_PER_STREAM, IDX_PER_STREAM), :],
                sem_tok,
            )
            for j in range(NSTREAM)
        ]
        title_copy = pltpu.async_copy(title_table.at[tid_v], trows_v,
                                      sem_title)

        for c in tok_copies:
            c.wait()
        title_copy.wait()

        def row_body(r, rcarry):
            accs = [rows_v[r * SEQ, pl.ds(q * 16, 16)] for q in range(4)]
            for t in range(1, SEQ):
                for q in range(4):
                    accs[q] = accs[q] + rows_v[r * SEQ + t, pl.ds(q * 16, 16)]
            v1 = tok_v[pl.ds(r * SEQ, 16)]
            v2 = tok_v[pl.ds(r * SEQ + 4, 16)]
            lanes = _lanes()
            nz1 = jnp.sum(jnp.where(v1 == 0, 1, 0))
            nz2 = jnp.sum(jnp.where((v2 == 0) & (lanes >= 12), 1, 0))
            nzf = (nz1 + nz2).astype(jnp.float32)
            nzs = jnp.zeros((16,), jnp.float32) + nzf
            invs = 1.0 / jnp.maximum(jnp.float32(SEQ) - nzs, 1.0)
            for q in range(4):
                r0 = row0_v[0, pl.ds(q * 16, 16)]
                outb_v[r, pl.ds(D + q * 16, 16)] = (accs[q] - nzs * r0) * invs
                outb_v[r, pl.ds(q * 16, 16)] = trows_v[r, pl.ds(q * 16, 16)]
            return rcarry

        lax.fori_loop(0, CHUNK, row_body, 0)
        pltpu.sync_copy(outb_v, out.at[pl.ds(base, CHUNK), :])
        return carry

    lax.fori_loop(0, NCHUNK, chunk_body, 0)


def kernel(title_ids, token_ids, title_table, token_table):
    tok_flat = token_ids.reshape(-1)
    return _movie_sc_kernel(title_ids, tok_flat, title_table, token_table)


# speedup vs baseline: 10.2840x; 1.2208x over previous
import functools

import jax
import jax.numpy as jnp
from jax import lax
from jax.experimental import pallas as pl
from jax.experimental.pallas import tpu as pltpu
from jax.experimental.pallas import tpu_sc as plsc

D = 64
SEQ = 20
B = 16384
NC = 2
NS = 16
NW = NC * NS
ROWS_PER_W = B // NW
CHUNK = 32
NCHUNK = ROWS_PER_W // CHUNK
TOK_PER_CHUNK = CHUNK * SEQ
IDX_PER_STREAM = 128
NSTREAM = TOK_PER_CHUNK // IDX_PER_STREAM


def _lanes():
    return lax.broadcasted_iota(jnp.int32, (16,), 0)


_mesh = plsc.VectorSubcoreMesh(core_axis_name="c", subcore_axis_name="s")


@functools.partial(
    pl.kernel,
    mesh=_mesh,
    compiler_params=pltpu.CompilerParams(needs_layout_passes=False,
                                         use_tc_tiling_on_sc=False),
    out_type=jax.ShapeDtypeStruct((B, 2 * D), jnp.float32),
    scratch_types=[
        pltpu.VMEM((2, TOK_PER_CHUNK), jnp.int32),
        pltpu.VMEM((2, CHUNK), jnp.int32),
        pltpu.VMEM((2, TOK_PER_CHUNK, D), jnp.float32),
        pltpu.VMEM((2, CHUNK, D), jnp.float32),
        pltpu.VMEM((2, CHUNK, 2 * D), jnp.float32),
        pltpu.VMEM((1, D), jnp.float32),
        pltpu.SemaphoreType.DMA,
        pltpu.SemaphoreType.DMA,
        pltpu.SemaphoreType.DMA,
        pltpu.SemaphoreType.DMA,
    ],
)
def _movie_sc_kernel(title_ids, tok_flat, title_table, token_table, out,
                     tok_v, tid_v, rows_v, trows_v, outb_v, row0_v,
                     sem_tok0, sem_tok1, sem_title0, sem_title1):
    wid = lax.axis_index("s") * NC + lax.axis_index("c")
    base0 = wid * ROWS_PER_W
    sem_tok = (sem_tok0, sem_tok1)
    sem_title = (sem_title0, sem_title1)

    pltpu.sync_copy(token_table.at[pl.ds(0, 1), :], row0_v)

    def stage_and_fire(c, b):
        base = base0 + c * CHUNK
        pltpu.sync_copy(tok_flat.at[pl.ds(base * SEQ, TOK_PER_CHUNK)],
                        tok_v.at[b])
        pltpu.sync_copy(title_ids.at[pl.ds(base, CHUNK)], tid_v.at[b])
        for j in range(NSTREAM):
            pltpu.async_copy(
                token_table.at[tok_v.at[b, pl.ds(j * IDX_PER_STREAM,
                                                 IDX_PER_STREAM)]],
                rows_v.at[b, pl.ds(j * IDX_PER_STREAM, IDX_PER_STREAM), :],
                sem_tok[b],
            )
        pltpu.async_copy(title_table.at[tid_v.at[b]], trows_v.at[b],
                         sem_title[b])

    def drain(b):
        pltpu.make_async_copy(token_table.at[pl.ds(0, TOK_PER_CHUNK), :],
                              rows_v.at[b], sem_tok[b]).wait()
        pltpu.make_async_copy(title_table.at[pl.ds(0, CHUNK), :],
                              trows_v.at[b], sem_title[b]).wait()

    def compute_and_write(c, b):
        base = base0 + c * CHUNK

        def row_body(r, rcarry):
            accs = [rows_v[b, r * SEQ, pl.ds(q * 16, 16)] for q in range(4)]
            for t in range(1, SEQ):
                for q in range(4):
                    accs[q] = accs[q] + rows_v[b, r * SEQ + t,
                                               pl.ds(q * 16, 16)]
            v1 = tok_v[b, pl.ds(r * SEQ, 16)]
            v2 = tok_v[b, pl.ds(r * SEQ + 4, 16)]
            lanes = _lanes()
            nz1 = jnp.sum(jnp.where(v1 == 0, 1, 0))
            nz2 = jnp.sum(jnp.where((v2 == 0) & (lanes >= 12), 1, 0))
            nzf = (nz1 + nz2).astype(jnp.float32)
            nzs = jnp.zeros((16,), jnp.float32) + nzf
            invs = 1.0 / jnp.maximum(jnp.float32(SEQ) - nzs, 1.0)
            for q in range(4):
                r0 = row0_v[0, pl.ds(q * 16, 16)]
                outb_v[b, r, pl.ds(D + q * 16, 16)] = \
                    (accs[q] - nzs * r0) * invs
                outb_v[b, r, pl.ds(q * 16, 16)] = \
                    trows_v[b, r, pl.ds(q * 16, 16)]
            return rcarry

        lax.fori_loop(0, CHUNK, row_body, 0)
        pltpu.sync_copy(outb_v.at[b], out.at[pl.ds(base, CHUNK), :])

    stage_and_fire(jnp.int32(0), 0)
    stage_and_fire(jnp.int32(1), 1)

    def pair_body(i, carry):
        for bb in range(2):
            c = i * 2 + bb
            drain(bb)
            compute_and_write(c, bb)

            @pl.when(c + 2 < NCHUNK)
            def _():
                stage_and_fire(c + 2, bb)
        return carry

    lax.fori_loop(0, NCHUNK // 2, pair_body, 0)


def kernel(title_ids, token_ids, title_table, token_table):
    tok_flat = token_ids.reshape(-1)
    return _movie_sc_kernel(title_ids, tok_flat, title_table, token_table)
